# baseline (speedup 1.0000x reference)
import jax
import jax.numpy as jnp
from jax import lax
from jax.experimental import pallas as pl
from jax.experimental.pallas import tpu as pltpu
from jax.experimental.pallas import tpu_sc as plsc

_C = 128
_NBUF = 4
_NW = 32
_BT = 10000


def _sc_onehot_t(idx_hbm, attr_out, idx_all, tail_idx, tail_buf, bufs, sems):
    n = idx_hbm.shape[0]
    k = attr_out.shape[0]
    tail = n % _C
    num_chunks = n // _C
    nbase = num_chunks // _NW
    rem = num_chunks - nbase * _NW
    wid = lax.axis_index("s") * 2 + lax.axis_index("c")

    my_chunks = nbase + jnp.where(wid < rem, 1, 0)
    chunk0 = nbase * wid + jnp.minimum(wid, rem)
    atom0 = chunk0 * _C

    pltpu.sync_copy(idx_hbm.at[pl.ds(atom0, nbase * _C)], idx_all.at[pl.ds(0, nbase * _C)])

    @pl.when(wid < rem)
    def _():
        pltpu.sync_copy(
            idx_hbm.at[pl.ds(atom0 + nbase * _C, _C)],
            idx_all.at[pl.ds(nbase * _C, _C)],
        )

    zeros = jnp.zeros((16,), jnp.float32)
    ones = jnp.full((16,), 1.0, jnp.float32)

    def zinit(r, _):
        for b in range(_NBUF):
            for c in range(_C // 16):
                bufs[b][r, pl.ds(c * 16, 16)] = zeros
        for c in range(tail // 16):
            tail_buf[r, pl.ds(c * 16, 16)] = zeros
        return 0

    lax.fori_loop(0, k, zinit, 0)

    def scatter(buf, j, val):
        for g in range(_C // 16):
            iv = idx_all[pl.ds(j * _C + g * 16, 16)]
            cols = lax.iota(jnp.int32, 16) + g * 16
            plsc.store_scatter(buf, [iv, cols], val)

    def dma(b, j):
        return pltpu.make_async_copy(
            bufs[b], attr_out.at[:, pl.ds((chunk0 + j) * _C, _C)], sems[b]
        )

    n_outer = (nbase + _NBUF) // _NBUF

    def outer(o, _):
        for b in range(_NBUF):
            j = o * _NBUF + b

            @pl.when(j < my_chunks)
            def _():
                @pl.when(o >= 1)
                def _():
                    dma(b, 0).wait()
                    scatter(bufs[b], j - _NBUF, zeros)

                scatter(bufs[b], j, ones)
                dma(b, j).start()

        return 0

    lax.fori_loop(0, n_outer, outer, 0)

    for b in range(_NBUF):
        @pl.when(b < my_chunks)
        def _():
            dma(b, 0).wait()

    @pl.when(wid == _NW - 1)
    def _():
        pltpu.sync_copy(idx_hbm.at[pl.ds(num_chunks * _C, tail)], tail_idx)
        for g in range(tail // 16):
            iv = tail_idx[pl.ds(g * 16, 16)]
            cols = lax.iota(jnp.int32, 16) + g * 16
            plsc.store_scatter(tail_buf, [iv, cols], ones)
        pltpu.sync_copy(tail_buf, attr_out.at[:, pl.ds(num_chunks * _C, tail)])


def _tc_fea(idx_ref, w_ref, out_ref):
    k = w_ref.shape[0]
    idxb = idx_ref[0].astype(jnp.int16)
    iota = lax.broadcasted_iota(jnp.int16, (k, _BT), 0)
    oh_t = (idxb == iota).astype(jnp.bfloat16)
    out_ref[...] = lax.dot_general(
        oh_t, w_ref[...].astype(jnp.bfloat16), (((0,), (0,)), ((), ())),
        preferred_element_type=jnp.float32,
    )


@jax.jit
def kernel(atom_number, W_embed):
    n = atom_number.shape[0]
    k, d = W_embed.shape
    assert n % _C == 32 and n % 16 == 0 and n % _BT == 0

    mesh = plsc.VectorSubcoreMesh(
        core_axis_name="c", subcore_axis_name="s", num_cores=2, num_subcores=16
    )
    nbase = (n // _C) // _NW
    attr_t = pl.kernel(
        _sc_onehot_t,
        out_type=jax.ShapeDtypeStruct((k, n), jnp.float32),
        mesh=mesh,
        compiler_params=pltpu.CompilerParams(needs_layout_passes=False),
        scratch_types=[
            pltpu.VMEM(((nbase + 1) * _C,), jnp.int32),
            pltpu.VMEM((n % _C,), jnp.int32),
            pltpu.VMEM((k, n % _C), jnp.float32),
            [pltpu.VMEM((k, _C), jnp.float32) for _ in range(_NBUF)],
            [pltpu.SemaphoreType.DMA for _ in range(_NBUF)],
        ],
    )(atom_number)

    nb = n // _BT
    fea = pl.pallas_call(
        _tc_fea,
        grid=(nb,),
        in_specs=[
            pl.BlockSpec((1, 1, _BT), lambda i: (i, 0, 0)),
            pl.BlockSpec((k, d), lambda i: (0, 0)),
        ],
        out_specs=pl.BlockSpec((_BT, d), lambda i: (i, 0)),
        out_shape=jax.ShapeDtypeStruct((n, d), jnp.float32),
    )(atom_number.reshape(nb, 1, _BT), W_embed)

    return attr_t.T, fea


# speedup vs baseline: 1.0483x; 1.0483x over previous
import jax
import jax.numpy as jnp
from jax import lax
from jax.experimental import pallas as pl
from jax.experimental.pallas import tpu as pltpu
from jax.experimental.pallas import tpu_sc as plsc

_C = 128
_NBUF = 4
_NW = 32
_BT = 20000


def _sc_onehot_t(idx_hbm, attr_out, idx_all, tail_idx, tail_buf, bufs, sems):
    n = idx_hbm.shape[0]
    k = attr_out.shape[0]
    tail = n % _C
    num_chunks = n // _C
    nbase = num_chunks // _NW
    rem = num_chunks - nbase * _NW
    wid = lax.axis_index("s") * 2 + lax.axis_index("c")

    my_chunks = nbase + jnp.where(wid < rem, 1, 0)
    chunk0 = nbase * wid + jnp.minimum(wid, rem)
    atom0 = chunk0 * _C

    pltpu.sync_copy(idx_hbm.at[pl.ds(atom0, nbase * _C)], idx_all.at[pl.ds(0, nbase * _C)])

    @pl.when(wid < rem)
    def _():
        pltpu.sync_copy(
            idx_hbm.at[pl.ds(atom0 + nbase * _C, _C)],
            idx_all.at[pl.ds(nbase * _C, _C)],
        )

    zeros = jnp.zeros((16,), jnp.float32)
    ones = jnp.full((16,), 1.0, jnp.float32)

    def zinit(r, _):
        for b in range(_NBUF):
            for c in range(_C // 16):
                bufs[b][r, pl.ds(c * 16, 16)] = zeros
        for c in range(tail // 16):
            tail_buf[r, pl.ds(c * 16, 16)] = zeros
        return 0

    lax.fori_loop(0, k, zinit, 0)

    def scatter(buf, j, val):
        for g in range(_C // 16):
            iv = idx_all[pl.ds(j * _C + g * 16, 16)]
            cols = lax.iota(jnp.int32, 16) + g * 16
            plsc.store_scatter(buf, [iv, cols], val)

    def dma(b, j):
        return pltpu.make_async_copy(
            bufs[b], attr_out.at[:, pl.ds((chunk0 + j) * _C, _C)], sems[b]
        )

    n_outer = (nbase + _NBUF) // _NBUF

    def outer(o, _):
        for b in range(_NBUF):
            j = o * _NBUF + b

            @pl.when(j < my_chunks)
            def _():
                @pl.when(o >= 1)
                def _():
                    dma(b, 0).wait()
                    scatter(bufs[b], j - _NBUF, zeros)

                scatter(bufs[b], j, ones)
                dma(b, j).start()

        return 0

    lax.fori_loop(0, n_outer, outer, 0)

    for b in range(_NBUF):
        @pl.when(b < my_chunks)
        def _():
            dma(b, 0).wait()

    @pl.when(wid == _NW - 1)
    def _():
        pltpu.sync_copy(idx_hbm.at[pl.ds(num_chunks * _C, tail)], tail_idx)
        for g in range(tail // 16):
            iv = tail_idx[pl.ds(g * 16, 16)]
            cols = lax.iota(jnp.int32, 16) + g * 16
            plsc.store_scatter(tail_buf, [iv, cols], ones)
        pltpu.sync_copy(tail_buf, attr_out.at[:, pl.ds(num_chunks * _C, tail)])


def _tc_fea(idx_ref, w_ref, out_ref):
    k = w_ref.shape[0]
    idxb = idx_ref[0].astype(jnp.int16)
    iota = lax.broadcasted_iota(jnp.int16, (k, _BT), 0)
    oh_t = (idxb == iota).astype(jnp.bfloat16)
    out_ref[...] = lax.dot_general(
        oh_t, w_ref[...].astype(jnp.bfloat16), (((0,), (0,)), ((), ())),
        preferred_element_type=jnp.float32,
    )


@jax.jit
def kernel(atom_number, W_embed):
    n = atom_number.shape[0]
    k, d = W_embed.shape
    assert n % _C == 32 and n % 16 == 0 and n % _BT == 0

    mesh = plsc.VectorSubcoreMesh(
        core_axis_name="c", subcore_axis_name="s", num_cores=2, num_subcores=16
    )
    nbase = (n // _C) // _NW
    attr_t = pl.kernel(
        _sc_onehot_t,
        out_type=jax.ShapeDtypeStruct((k, n), jnp.float32),
        mesh=mesh,
        compiler_params=pltpu.CompilerParams(needs_layout_passes=False),
        scratch_types=[
            pltpu.VMEM(((nbase + 1) * _C,), jnp.int32),
            pltpu.VMEM((n % _C,), jnp.int32),
            pltpu.VMEM((k, n % _C), jnp.float32),
            [pltpu.VMEM((k, _C), jnp.float32) for _ in range(_NBUF)],
            [pltpu.SemaphoreType.DMA for _ in range(_NBUF)],
        ],
    )(atom_number)

    nb = n // _BT
    fea = pl.pallas_call(
        _tc_fea,
        grid=(nb,),
        in_specs=[
            pl.BlockSpec((1, 1, _BT), lambda i: (i, 0, 0)),
            pl.BlockSpec((k, d), lambda i: (0, 0)),
        ],
        out_specs=pl.BlockSpec((_BT, d), lambda i: (i, 0)),
        out_shape=jax.ShapeDtypeStruct((n, d), jnp.float32),
    )(atom_number.reshape(nb, 1, _BT), W_embed)

    return attr_t.T, fea
